# baseline (speedup 1.0000x reference)
import functools

import jax
import jax.numpy as jnp
from jax import lax
from jax.experimental import pallas as pl
from jax.experimental.pallas import tpu as pltpu
import jax.experimental.pallas.tpu_sc as plsc

NUM_USERS = 50000
NUM_ITEMS = 50000
N = NUM_USERS + NUM_ITEMS
D = 32
DH = D // 2
NNZ = 1600000
BATCH = 16384

NC = 2
NS = 16
ACC_ROWS = 100352
E_TILE = NNZ // NS
E_STEP = 80
N_STEPS = E_TILE // E_STEP
NBUF = 5
M_STEPS = 5
MB = M_STEPS * E_STEP
N_BLOCKS = N_STEPS // M_STEPS
ZCH = 64
CCH = 40

_mesh = lambda: plsc.VectorSubcoreMesh(
    core_axis_name="c", subcore_axis_name="s", num_cores=NC, num_subcores=NS)


def _tc_normalize(x):
    R = x.shape[0]
    BLK = 2000

    def body(x_ref, lo_ref, hi_ref):
        v = x_ref[...]
        n = jnp.sqrt(jnp.sum(v * v, axis=1, keepdims=True))
        v = v / jnp.maximum(n, 1e-12)
        lo_ref[...] = v[:, :DH]
        hi_ref[...] = v[:, DH:]

    return pl.pallas_call(
        body,
        grid=(R // BLK,),
        in_specs=[pl.BlockSpec((BLK, D), lambda i: (i, 0))],
        out_specs=[pl.BlockSpec((BLK, DH), lambda i: (i, 0)),
                   pl.BlockSpec((BLK, DH), lambda i: (i, 0))],
        out_shape=[jax.ShapeDtypeStruct((R, DH), jnp.float32),
                   jax.ShapeDtypeStruct((R, DH), jnp.float32)],
    )(x)


def _sc_layer(ego, gcol, grow, gval):
    @functools.partial(
        pl.kernel,
        out_type=jax.ShapeDtypeStruct((2 * N, DH), jnp.float32),
        mesh=_mesh(),
        scratch_types=(
            [pltpu.VMEM((E_STEP,), jnp.int32)] * NBUF
            + [pltpu.VMEM((E_STEP,), jnp.int32)] * NBUF
            + [pltpu.VMEM((E_STEP, DH), jnp.float32)] * NBUF
            + [pltpu.VMEM((MB,), jnp.int32)] * 2
            + [pltpu.VMEM((MB,), jnp.int32)] * 2
            + [pltpu.VMEM((MB,), jnp.float32)] * 2
            + [
                pltpu.VMEM((ZCH, DH), jnp.float32),
                pltpu.VMEM((CCH, DH), jnp.float32),
                pltpu.VMEM((CCH, DH), jnp.float32),
                pltpu.VMEM_SHARED((ACC_ROWS, DH), jnp.float32),
                pltpu.SemaphoreType.DMA,
                pltpu.SemaphoreType.DMA,
                pltpu.SemaphoreType.DMA,
            ]
        ),
        compiler_params=pltpu.CompilerParams(use_tc_tiling_on_sc=False),
    )
    def k(ego_hbm, gcol_hbm, grow_hbm, gval_hbm, out_hbm,
          c0, c1, c2, c3, c4, l0, l1, l2, l3, l4,
          m0, m1, m2, m3, m4, cb0, cb1, rb0, rb1, vb0, vb1,
          zv, aggv, egov, acc, gsem, msem, ssem):
        colv = [c0, c1, c2, c3, c4]
        rowloc = [l0, l1, l2, l3, l4]
        msg = [m0, m1, m2, m3, m4]
        colB = [cb0, cb1]
        rowB = [rb0, rb1]
        valB = [vb0, vb1]
        c = lax.axis_index("c")
        s = lax.axis_index("s")
        half_base = c * N

        zero = jnp.zeros((16,), jnp.float32)
        for g in range(ZCH):
            zv[g, pl.ds(0, 16)] = zero
        rows_per_tile = ACC_ROWS // NS

        def zbody(i, carry):
            r0 = s * rows_per_tile + i * ZCH
            pltpu.sync_copy(zv, acc.at[pl.ds(r0, ZCH)])
            return carry

        lax.fori_loop(0, rows_per_tile // ZCH, zbody, 0)
        plsc.subcore_barrier()

        def macro_src(bm, p):
            e0 = s * E_TILE + bm * MB
            return (gcol_hbm.at[pl.ds(e0, MB)],
                    grow_hbm.at[pl.ds(e0, MB)],
                    gval_hbm.at[pl.ds(e0, MB)])

        def issue_macro(bm, p):
            cs, rs, vs = macro_src(bm, p)
            pltpu.async_copy(cs, colB[p], msem)
            pltpu.async_copy(rs, rowB[p], msem)
            pltpu.async_copy(vs, valB[p], msem)

        def wait_macro(bm, p):
            cs, rs, vs = macro_src(bm, p)
            pltpu.make_async_copy(cs, colB[p], msem).wait()
            pltpu.make_async_copy(rs, rowB[p], msem).wait()
            pltpu.make_async_copy(vs, valB[p], msem).wait()

        def fill_colv(p, j, w):
            for g in range(E_STEP // 16):
                colv[w][pl.ds(g * 16, 16)] = (
                    colB[p][pl.ds(j * E_STEP + g * 16, 16)] + half_base)

        def drain_scatter(b):
            pltpu.make_async_copy(msg[b], acc.at[rowloc[b]], ssem).wait()

        def compute(p, j, u):
            mref = msg[u]
            for g in range(E_STEP // 16):
                sl = pl.ds(j * E_STEP + g * 16, 16)
                rowloc[u][pl.ds(g * 16, 16)] = rowB[p][sl]
                vv = valB[p][sl]
                for e in range(16):
                    v = vv[e]
                    idx = g * 16 + e
                    mref[idx, pl.ds(0, 16)] = mref[idx, pl.ds(0, 16)] * v

        cs0, rs0, vs0 = macro_src(0, 0)
        pltpu.sync_copy(cs0, colB[0])
        pltpu.sync_copy(rs0, rowB[0])
        pltpu.sync_copy(vs0, valB[0])
        for j0 in range(3):
            fill_colv(0, j0, j0)
            pltpu.async_copy(ego_hbm.at[colv[j0]], msg[j0], gsem)

        def block2(i2, carry):
            for p in (0, 1):
                bm = i2 * 2 + p
                for j in range(M_STEPS):
                    u = j % NBUF
                    kk = bm * M_STEPS + j
                    if j == 0:
                        @pl.when(bm + 1 < N_BLOCKS)
                        def _():
                            issue_macro(bm + 1, (p + 1) % 2)
                    pltpu.make_async_copy(
                        ego_hbm.at[colv[u]], msg[u], gsem).wait()
                    compute(p, j, u)
                    wprev = (u + NBUF - 1) % NBUF

                    @pl.when(kk > 0)
                    def _():
                        drain_scatter(wprev)

                    pltpu.async_copy(msg[u], acc.at[rowloc[u]], ssem,
                                     add=True)
                    if j == M_STEPS - 3:
                        @pl.when(bm + 1 < N_BLOCKS)
                        def _():
                            wait_macro(bm + 1, (p + 1) % 2)
                    j3 = j + 3
                    w = (u + 3) % NBUF
                    if j3 < M_STEPS:
                        fill_colv(p, j3, w)
                        pltpu.async_copy(ego_hbm.at[colv[w]], msg[w], gsem)
                    else:
                        @pl.when(bm + 1 < N_BLOCKS)
                        def _():
                            fill_colv((p + 1) % 2, j3 - M_STEPS, w)
                            pltpu.async_copy(ego_hbm.at[colv[w]], msg[w],
                                             gsem)
            return carry

        lax.fori_loop(0, N_BLOCKS // 2, block2, 0)
        drain_scatter((N_STEPS - 1) % NBUF)
        plsc.subcore_barrier()

        nch = N // CCH
        my_n = nch // NS + jnp.where(s < nch % NS, 1, 0)

        def cbody(i, carry):
            lr0 = (s + i * NS) * CCH
            gr0 = half_base + lr0
            pltpu.sync_copy(acc.at[pl.ds(lr0, CCH)], aggv)
            pltpu.sync_copy(ego_hbm.at[pl.ds(gr0, CCH)], egov)
            for g in range(CCH):
                sl = pl.ds(0, 16)
                a = aggv[g, sl]
                aggv[g, sl] = a + a * egov[g, sl]
            pltpu.sync_copy(aggv, out_hbm.at[pl.ds(gr0, CCH)])
            return carry

        lax.fori_loop(0, my_n, cbody, 0)

    return k(ego, gcol, grow, gval)


def _sc_gather_mean(x, e1, e2, e3, user_indices, item_indices):
    NW = NC * NS
    per_w = BATCH // NW
    GSTEP = 128

    @functools.partial(
        pl.kernel,
        out_type=[jax.ShapeDtypeStruct((BATCH, DH), jnp.float32)
                  for _ in range(4)],
        mesh=_mesh(),
        scratch_types=[
            pltpu.VMEM((GSTEP,), jnp.int32),
            pltpu.VMEM((GSTEP,), jnp.int32),
            pltpu.VMEM((GSTEP, DH), jnp.float32),
            pltpu.VMEM((GSTEP, DH), jnp.float32),
            pltpu.VMEM((GSTEP, DH), jnp.float32),
            pltpu.VMEM((GSTEP, DH), jnp.float32),
            pltpu.SemaphoreType.DMA,
        ],
        compiler_params=pltpu.CompilerParams(use_tc_tiling_on_sc=False),
    )
    def k(x_hbm, e1_hbm, e2_hbm, e3_hbm, ui_hbm, ii_hbm,
          ulo_out, uhi_out, ilo_out, ihi_out,
          idxv, idxg, b0, b1, b2, b3, sem):
        c = lax.axis_index("c")
        s = lax.axis_index("s")
        wid = s * NC + c
        outs = {(0, 0): ulo_out, (0, 1): uhi_out,
                (1, 0): ilo_out, (1, 1): ihi_out}

        def body(i, carry):
            r0 = wid * per_w + i * GSTEP
            for li, idx_hbm in ((0, ui_hbm), (1, ii_hbm)):
                pltpu.sync_copy(idx_hbm.at[pl.ds(r0, GSTEP)], idxv)
                for hh in (0, 1):
                    off = li * NUM_USERS + hh * N
                    for g in range(GSTEP // 16):
                        sl = pl.ds(g * 16, 16)
                        idxg[sl] = idxv[sl] + off
                    pltpu.async_copy(x_hbm.at[idxg], b0, sem).wait()
                    pltpu.async_copy(e1_hbm.at[idxg], b1, sem).wait()
                    pltpu.async_copy(e2_hbm.at[idxg], b2, sem).wait()
                    pltpu.async_copy(e3_hbm.at[idxg], b3, sem).wait()
                    for g in range(GSTEP):
                        sl = pl.ds(0, 16)
                        b0[g, sl] = (b0[g, sl] + b1[g, sl]
                                     + b2[g, sl] + b3[g, sl]) * 0.25
                    pltpu.sync_copy(b0, outs[(li, hh)].at[pl.ds(r0, GSTEP)])
            return carry

        lax.fori_loop(0, per_w // GSTEP, body, 0)

    return k(x, e1, e2, e3, user_indices, item_indices)


def _tc_head(u_lo, u_hi, i_lo, i_hi, Wa, ba, W1, b1, W2, b2):
    BLK = 2048

    def body(ulo_ref, uhi_ref, ilo_ref, ihi_ref,
             wa_ref, ba_ref, w1_ref, b1_ref, w2_ref, b2_ref, o_ref):
        u = jnp.concatenate([ulo_ref[...], uhi_ref[...]], axis=1)
        it = jnp.concatenate([ilo_ref[...], ihi_ref[...]], axis=1)
        u = u / jnp.maximum(
            jnp.sqrt(jnp.sum(u * u, axis=1, keepdims=True)), 1e-12)
        it = it / jnp.maximum(
            jnp.sqrt(jnp.sum(it * it, axis=1, keepdims=True)), 1e-12)
        mf = u * it
        cat = jnp.concatenate([u, it], axis=1)
        logits = jnp.dot(mf, wa_ref[...],
                         preferred_element_type=jnp.float32) + ba_ref[...]
        h = jnp.maximum(
            jnp.dot(cat, w1_ref[...],
                    preferred_element_type=jnp.float32) + b1_ref[...], 0.0)
        mlp = jnp.dot(h, w2_ref[...],
                      preferred_element_type=jnp.float32) + b2_ref[...]
        o_ref[...] = jax.nn.sigmoid(logits + mlp)

    zmap = lambda i: (0, 0)
    hmap = lambda i: (i, 0)
    return pl.pallas_call(
        body,
        grid=(BATCH // BLK,),
        in_specs=[
            pl.BlockSpec((BLK, DH), hmap),
            pl.BlockSpec((BLK, DH), hmap),
            pl.BlockSpec((BLK, DH), hmap),
            pl.BlockSpec((BLK, DH), hmap),
            pl.BlockSpec((D, 1), zmap),
            pl.BlockSpec((1, 1), zmap),
            pl.BlockSpec((2 * D, 4 * D), zmap),
            pl.BlockSpec((1, 4 * D), zmap),
            pl.BlockSpec((4 * D, 1), zmap),
            pl.BlockSpec((1, 1), zmap),
        ],
        out_specs=pl.BlockSpec((BLK, 1), lambda i: (i, 0)),
        out_shape=jax.ShapeDtypeStruct((BATCH, 1), jnp.float32),
    )(u_lo, u_hi, i_lo, i_hi, Wa, ba.reshape(1, 1), W1,
      b1.reshape(1, 4 * D), W2, b2.reshape(1, 1))


def kernel(user_emb, item_emb, graph_val, Wa, ba, W1, b1, W2, b2,
           graph_idx, user_indices, item_indices):
    u_lo, u_hi = _tc_normalize(user_emb)
    i_lo, i_hi = _tc_normalize(item_emb)
    x = jnp.concatenate([u_lo, i_lo, u_hi, i_hi], axis=0)
    gcol = graph_idx[1]
    grow = graph_idx[0]
    e1 = _sc_layer(x, gcol, grow, graph_val)
    e2 = _sc_layer(e1, gcol, grow, graph_val)
    e3 = _sc_layer(e2, gcol, grow, graph_val)
    g_ulo, g_uhi, g_ilo, g_ihi = _sc_gather_mean(
        x, e1, e2, e3, user_indices, item_indices)
    return _tc_head(g_ulo, g_uhi, g_ilo, g_ihi, Wa, ba, W1, b1, W2, b2)


# speedup vs baseline: 1.0490x; 1.0490x over previous
import functools

import jax
import jax.numpy as jnp
from jax import lax
from jax.experimental import pallas as pl
from jax.experimental.pallas import tpu as pltpu
import jax.experimental.pallas.tpu_sc as plsc

NUM_USERS = 50000
NUM_ITEMS = 50000
N = NUM_USERS + NUM_ITEMS
D = 32
NNZ = 1600000
BATCH = 16384

NC = 2
NS = 16
H = N // NC
ACC_ROWS = 51200
DUMMY = H
E_STEP = 128
N_GSTEPS = NNZ // E_STEP
T_STEPS = 785
NBUF = 5
ZCH = 64
CCH = 40

_mesh = lambda: plsc.VectorSubcoreMesh(
    core_axis_name="c", subcore_axis_name="s", num_cores=NC, num_subcores=NS)


def _tc_normalize(x):
    R = x.shape[0]
    BLK = 2000

    def body(x_ref, o_ref):
        v = x_ref[...]
        n = jnp.sqrt(jnp.sum(v * v, axis=1, keepdims=True))
        o_ref[...] = v / jnp.maximum(n, 1e-12)

    return pl.pallas_call(
        body,
        grid=(R // BLK,),
        in_specs=[pl.BlockSpec((BLK, D), lambda i: (i, 0))],
        out_specs=pl.BlockSpec((BLK, D), lambda i: (i, 0)),
        out_shape=jax.ShapeDtypeStruct((R, D), jnp.float32),
    )(x)


def _sc_layer(ego, gcol, grow, gval):
    @functools.partial(
        pl.kernel,
        out_type=jax.ShapeDtypeStruct((N, D), jnp.float32),
        mesh=_mesh(),
        scratch_types=(
            [pltpu.VMEM((E_STEP,), jnp.int32)] * NBUF
            + [pltpu.VMEM((E_STEP,), jnp.int32)] * NBUF
            + [pltpu.VMEM((E_STEP,), jnp.float32)] * NBUF
            + [pltpu.VMEM((E_STEP, D), jnp.float32)] * NBUF
            + [
                pltpu.VMEM((ZCH, D), jnp.float32),
                pltpu.VMEM((CCH, D), jnp.float32),
                pltpu.VMEM((CCH, D), jnp.float32),
                pltpu.VMEM_SHARED((ACC_ROWS, D), jnp.float32),
                pltpu.SemaphoreType.DMA,
                pltpu.SemaphoreType.DMA,
                pltpu.SemaphoreType.DMA,
            ]
        ),
        compiler_params=pltpu.CompilerParams(use_tc_tiling_on_sc=False),
    )
    def k(ego_hbm, gcol_hbm, grow_hbm, gval_hbm, out_hbm,
          c0, c1, c2, c3, c4, r0, r1, r2, r3, r4,
          v0, v1, v2, v3, v4, m0, m1, m2, m3, m4,
          zv, aggv, egov, acc, gsem, msem, ssem):
        colv = [c0, c1, c2, c3, c4]
        rowv = [r0, r1, r2, r3, r4]
        valv = [v0, v1, v2, v3, v4]
        msg = [m0, m1, m2, m3, m4]
        c = lax.axis_index("c")
        s = lax.axis_index("s")
        row_base = c * H

        zero = jnp.zeros((16,), jnp.float32)
        for g in range(ZCH):
            for h in range(D // 16):
                zv[g, pl.ds(h * 16, 16)] = zero
        rows_per_tile = ACC_ROWS // NS

        def zbody(i, carry):
            r0 = s * rows_per_tile + i * ZCH
            pltpu.sync_copy(zv, acc.at[pl.ds(r0, ZCH)])
            return carry

        lax.fori_loop(0, rows_per_tile // ZCH, zbody, 0)
        plsc.subcore_barrier()

        def gstep(kk):
            gs = s + kk * NS
            ok = gs < N_GSTEPS
            return jnp.where(ok, gs, 0), ok

        def meta_src(kk):
            gs, _ = gstep(kk)
            e0 = gs * E_STEP
            return (gcol_hbm.at[pl.ds(e0, E_STEP)],
                    grow_hbm.at[pl.ds(e0, E_STEP)],
                    gval_hbm.at[pl.ds(e0, E_STEP)])

        def issue_meta(kk, b):
            cs, rs, vs = meta_src(kk)
            pltpu.async_copy(cs, colv[b], msem)
            pltpu.async_copy(rs, rowv[b], msem)
            pltpu.async_copy(vs, valv[b], msem)

        def wait_meta(kk, b):
            cs, rs, vs = meta_src(kk)
            pltpu.make_async_copy(cs, colv[b], msem).wait()
            pltpu.make_async_copy(rs, rowv[b], msem).wait()
            pltpu.make_async_copy(vs, valv[b], msem).wait()

        def drain_scatter(b):
            pltpu.make_async_copy(msg[b], acc.at[rowv[b]], ssem).wait()

        def compute(kk, u):
            _, ok_step = gstep(kk)
            vfac = jnp.where(ok_step, 1.0, 0.0).astype(jnp.float32)
            mref = msg[u]
            for g in range(E_STEP // 16):
                sl = pl.ds(g * 16, 16)
                r = rowv[u][sl]
                lr = r - row_base
                ok = (lr >= 0) & (lr < H)
                dummy = DUMMY + s * 16 + lax.iota(jnp.int32, 16)
                rowv[u][sl] = jnp.where(ok, lr, dummy)
                vv = valv[u][sl] * vfac
                for e in range(16):
                    v = vv[e]
                    idx = g * 16 + e
                    mref[idx, pl.ds(0, 16)] = mref[idx, pl.ds(0, 16)] * v
                    mref[idx, pl.ds(16, 16)] = mref[idx, pl.ds(16, 16)] * v

        for j0 in range(3):
            cs, rs, vs = meta_src(j0)
            pltpu.sync_copy(cs, colv[j0])
            pltpu.sync_copy(rs, rowv[j0])
            pltpu.sync_copy(vs, valv[j0])
            pltpu.async_copy(ego_hbm.at[colv[j0]], msg[j0], gsem)
        issue_meta(3, 3)

        def block(i, carry):
            for u in range(NBUF):
                kk = i * NBUF + u
                pltpu.make_async_copy(
                    ego_hbm.at[colv[u]], msg[u], gsem).wait()
                compute(kk, u)
                wprev = (u + NBUF - 1) % NBUF

                @pl.when(kk > 0)
                def _():
                    drain_scatter(wprev)

                pltpu.async_copy(msg[u], acc.at[rowv[u]], ssem, add=True)
                w = (u + 3) % NBUF

                @pl.when(kk + 3 < T_STEPS)
                def _():
                    wait_meta(kk + 3, w)
                    pltpu.async_copy(ego_hbm.at[colv[w]], msg[w], gsem)

                w2 = (u + 4) % NBUF

                @pl.when(kk + 4 < T_STEPS)
                def _():
                    issue_meta(kk + 4, w2)
            return carry

        lax.fori_loop(0, T_STEPS // NBUF, block, 0)
        drain_scatter((T_STEPS - 1) % NBUF)
        plsc.subcore_barrier()

        nch = H // CCH
        my_n = nch // NS + jnp.where(s < nch % NS, 1, 0)

        def cbody(i, carry):
            lr0 = (s + i * NS) * CCH
            gr0 = row_base + lr0
            pltpu.sync_copy(acc.at[pl.ds(lr0, CCH)], aggv)
            pltpu.sync_copy(ego_hbm.at[pl.ds(gr0, CCH)], egov)
            for g in range(CCH):
                for h in range(D // 16):
                    sl = pl.ds(h * 16, 16)
                    a = aggv[g, sl]
                    aggv[g, sl] = a + a * egov[g, sl]
            pltpu.sync_copy(aggv, out_hbm.at[pl.ds(gr0, CCH)])
            return carry

        lax.fori_loop(0, my_n, cbody, 0)

    return k(ego, gcol, grow, gval)


def _sc_gather_mean(x, e1, e2, e3, user_indices, item_indices):
    NW = NC * NS
    per_w = BATCH // NW
    GSTEP = 128

    @functools.partial(
        pl.kernel,
        out_type=[
            jax.ShapeDtypeStruct((BATCH, D), jnp.float32),
            jax.ShapeDtypeStruct((BATCH, D), jnp.float32),
        ],
        mesh=_mesh(),
        scratch_types=[
            pltpu.VMEM((GSTEP,), jnp.int32),
            pltpu.VMEM((GSTEP, D), jnp.float32),
            pltpu.VMEM((GSTEP, D), jnp.float32),
            pltpu.VMEM((GSTEP, D), jnp.float32),
            pltpu.VMEM((GSTEP, D), jnp.float32),
            pltpu.SemaphoreType.DMA,
        ],
        compiler_params=pltpu.CompilerParams(use_tc_tiling_on_sc=False),
    )
    def k(x_hbm, e1_hbm, e2_hbm, e3_hbm, ui_hbm, ii_hbm, u_out, i_out,
          idxv, b0, b1, b2, b3, sem):
        c = lax.axis_index("c")
        s = lax.axis_index("s")
        wid = s * NC + c

        def make_body(idx_hbm, out_hbm, off):
            def body(i, carry):
                r0 = wid * per_w + i * GSTEP
                pltpu.sync_copy(idx_hbm.at[pl.ds(r0, GSTEP)], idxv)
                if off:
                    for g in range(GSTEP // 16):
                        sl = pl.ds(g * 16, 16)
                        idxv[sl] = idxv[sl] + NUM_USERS
                pltpu.async_copy(x_hbm.at[idxv], b0, sem).wait()
                pltpu.async_copy(e1_hbm.at[idxv], b1, sem).wait()
                pltpu.async_copy(e2_hbm.at[idxv], b2, sem).wait()
                pltpu.async_copy(e3_hbm.at[idxv], b3, sem).wait()
                for g in range(GSTEP):
                    for h in range(D // 16):
                        sl = pl.ds(h * 16, 16)
                        b0[g, sl] = (b0[g, sl] + b1[g, sl]
                                     + b2[g, sl] + b3[g, sl]) * 0.25
                pltpu.sync_copy(b0, out_hbm.at[pl.ds(r0, GSTEP)])
                return carry
            return body

        lax.fori_loop(0, per_w // GSTEP, make_body(ui_hbm, u_out, False), 0)
        lax.fori_loop(0, per_w // GSTEP, make_body(ii_hbm, i_out, True), 0)

    return k(x, e1, e2, e3, user_indices, item_indices)


def _tc_head(u_raw, it_raw, Wa, ba, W1, b1, W2, b2):
    BLK = 2048

    def body(u_ref, i_ref, wa_ref, ba_ref, w1_ref, b1_ref, w2_ref, b2_ref,
             o_ref):
        u = u_ref[...]
        it = i_ref[...]
        u = u / jnp.maximum(
            jnp.sqrt(jnp.sum(u * u, axis=1, keepdims=True)), 1e-12)
        it = it / jnp.maximum(
            jnp.sqrt(jnp.sum(it * it, axis=1, keepdims=True)), 1e-12)
        mf = u * it
        cat = jnp.concatenate([u, it], axis=1)
        logits = jnp.dot(mf, wa_ref[...],
                         preferred_element_type=jnp.float32) + ba_ref[...]
        h = jnp.maximum(
            jnp.dot(cat, w1_ref[...],
                    preferred_element_type=jnp.float32) + b1_ref[...], 0.0)
        mlp = jnp.dot(h, w2_ref[...],
                      preferred_element_type=jnp.float32) + b2_ref[...]
        o_ref[...] = jax.nn.sigmoid(logits + mlp)

    zmap = lambda i: (0, 0)
    return pl.pallas_call(
        body,
        grid=(BATCH // BLK,),
        in_specs=[
            pl.BlockSpec((BLK, D), lambda i: (i, 0)),
            pl.BlockSpec((BLK, D), lambda i: (i, 0)),
            pl.BlockSpec((D, 1), zmap),
            pl.BlockSpec((1, 1), zmap),
            pl.BlockSpec((2 * D, 4 * D), zmap),
            pl.BlockSpec((1, 4 * D), zmap),
            pl.BlockSpec((4 * D, 1), zmap),
            pl.BlockSpec((1, 1), zmap),
        ],
        out_specs=pl.BlockSpec((BLK, 1), lambda i: (i, 0)),
        out_shape=jax.ShapeDtypeStruct((BATCH, 1), jnp.float32),
    )(u_raw, it_raw, Wa, ba.reshape(1, 1), W1, b1.reshape(1, 4 * D), W2,
      b2.reshape(1, 1))


def kernel(user_emb, item_emb, graph_val, Wa, ba, W1, b1, W2, b2,
           graph_idx, user_indices, item_indices):
    xu = _tc_normalize(user_emb)
    xi = _tc_normalize(item_emb)
    x = jnp.concatenate([xu, xi], axis=0)
    gcol = graph_idx[1]
    grow = graph_idx[0]
    e1 = _sc_layer(x, gcol, grow, graph_val)
    e2 = _sc_layer(e1, gcol, grow, graph_val)
    e3 = _sc_layer(e2, gcol, grow, graph_val)
    u_raw, it_raw = _sc_gather_mean(x, e1, e2, e3, user_indices, item_indices)
    return _tc_head(u_raw, it_raw, Wa, ba, W1, b1, W2, b2)


# speedup vs baseline: 1.0805x; 1.0301x over previous
import functools

import jax
import jax.numpy as jnp
from jax import lax
from jax.experimental import pallas as pl
from jax.experimental.pallas import tpu as pltpu
import jax.experimental.pallas.tpu_sc as plsc

NUM_USERS = 50000
NUM_ITEMS = 50000
N = NUM_USERS + NUM_ITEMS
D = 32
NNZ = 1600000
BATCH = 16384

NC = 2
NS = 16
H = N // NC
ACC_ROWS = 51200
DUMMY = H
E_TILE = NNZ // NS
E_STEP = 80
N_STEPS = E_TILE // E_STEP
NBUF = 5
M_STEPS = 5
MB = M_STEPS * E_STEP
N_BLOCKS = N_STEPS // M_STEPS
ZCH = 64
CCH = 40

_mesh = lambda: plsc.VectorSubcoreMesh(
    core_axis_name="c", subcore_axis_name="s", num_cores=NC, num_subcores=NS)


def _tc_normalize(x):
    R = x.shape[0]
    BLK = 2000

    def body(x_ref, o_ref):
        v = x_ref[...]
        n = jnp.sqrt(jnp.sum(v * v, axis=1, keepdims=True))
        o_ref[...] = v / jnp.maximum(n, 1e-12)

    return pl.pallas_call(
        body,
        grid=(R // BLK,),
        in_specs=[pl.BlockSpec((BLK, D), lambda i: (i, 0))],
        out_specs=pl.BlockSpec((BLK, D), lambda i: (i, 0)),
        out_shape=jax.ShapeDtypeStruct((R, D), jnp.float32),
    )(x)


def _sc_layer(ego, gcol, grow, gval):
    @functools.partial(
        pl.kernel,
        out_type=jax.ShapeDtypeStruct((N, D), jnp.float32),
        mesh=_mesh(),
        scratch_types=(
            [pltpu.VMEM((E_STEP,), jnp.int32)] * NBUF
            + [pltpu.VMEM((E_STEP,), jnp.int32)] * NBUF
            + [pltpu.VMEM((E_STEP, D), jnp.float32)] * NBUF
            + [pltpu.VMEM((MB,), jnp.int32)] * 2
            + [pltpu.VMEM((MB,), jnp.int32)] * 2
            + [pltpu.VMEM((MB,), jnp.float32)] * 2
            + [
                pltpu.VMEM((ZCH, D), jnp.float32),
                pltpu.VMEM((CCH, D), jnp.float32),
                pltpu.VMEM((CCH, D), jnp.float32),
                pltpu.VMEM_SHARED((ACC_ROWS, D), jnp.float32),
                pltpu.SemaphoreType.DMA,
                pltpu.SemaphoreType.DMA,
                pltpu.SemaphoreType.DMA,
            ]
        ),
        compiler_params=pltpu.CompilerParams(use_tc_tiling_on_sc=False),
    )
    def k(ego_hbm, gcol_hbm, grow_hbm, gval_hbm, out_hbm,
          c0, c1, c2, c3, c4, l0, l1, l2, l3, l4,
          m0, m1, m2, m3, m4, cb0, cb1, rb0, rb1, vb0, vb1,
          zv, aggv, egov, acc, gsem, msem, ssem):
        colv = [c0, c1, c2, c3, c4]
        rowloc = [l0, l1, l2, l3, l4]
        msg = [m0, m1, m2, m3, m4]
        colB = [cb0, cb1]
        rowB = [rb0, rb1]
        valB = [vb0, vb1]
        c = lax.axis_index("c")
        s = lax.axis_index("s")
        row_base = c * H

        zero = jnp.zeros((16,), jnp.float32)
        for g in range(ZCH):
            for h in range(D // 16):
                zv[g, pl.ds(h * 16, 16)] = zero
        rows_per_tile = ACC_ROWS // NS

        def zbody(i, carry):
            r0 = s * rows_per_tile + i * ZCH
            pltpu.sync_copy(zv, acc.at[pl.ds(r0, ZCH)])
            return carry

        lax.fori_loop(0, rows_per_tile // ZCH, zbody, 0)
        plsc.subcore_barrier()

        def macro_src(bm, p):
            e0 = s * E_TILE + bm * MB
            return (gcol_hbm.at[pl.ds(e0, MB)],
                    grow_hbm.at[pl.ds(e0, MB)],
                    gval_hbm.at[pl.ds(e0, MB)])

        def issue_macro(bm, p):
            cs, rs, vs = macro_src(bm, p)
            pltpu.async_copy(cs, colB[p], msem)
            pltpu.async_copy(rs, rowB[p], msem)
            pltpu.async_copy(vs, valB[p], msem)

        def wait_macro(bm, p):
            cs, rs, vs = macro_src(bm, p)
            pltpu.make_async_copy(cs, colB[p], msem).wait()
            pltpu.make_async_copy(rs, rowB[p], msem).wait()
            pltpu.make_async_copy(vs, valB[p], msem).wait()

        def fill_colv(p, j, w):
            for g in range(E_STEP // 16):
                colv[w][pl.ds(g * 16, 16)] = (
                    colB[p][pl.ds(j * E_STEP + g * 16, 16)])

        def drain_scatter(b):
            pltpu.make_async_copy(msg[b], acc.at[rowloc[b]], ssem).wait()

        def compute(p, j, u):
            mref = msg[u]
            for g in range(E_STEP // 16):
                sl = pl.ds(j * E_STEP + g * 16, 16)
                r = rowB[p][sl]
                lr = r - row_base
                ok = (lr >= 0) & (lr < H)
                dummy = DUMMY + s * 16 + lax.iota(jnp.int32, 16)
                rowloc[u][pl.ds(g * 16, 16)] = jnp.where(ok, lr, dummy)
                vv = valB[p][sl]
                for e in range(16):
                    v = vv[e]
                    idx = g * 16 + e
                    mref[idx, pl.ds(0, 16)] = mref[idx, pl.ds(0, 16)] * v
                    mref[idx, pl.ds(16, 16)] = mref[idx, pl.ds(16, 16)] * v

        cs0, rs0, vs0 = macro_src(0, 0)
        pltpu.sync_copy(cs0, colB[0])
        pltpu.sync_copy(rs0, rowB[0])
        pltpu.sync_copy(vs0, valB[0])
        for j0 in range(3):
            fill_colv(0, j0, j0)
            pltpu.async_copy(ego_hbm.at[colv[j0]], msg[j0], gsem)

        def block2(i2, carry):
            for p in (0, 1):
                bm = i2 * 2 + p
                for j in range(M_STEPS):
                    u = j % NBUF
                    kk = bm * M_STEPS + j
                    if j == 0:
                        @pl.when(bm + 1 < N_BLOCKS)
                        def _():
                            issue_macro(bm + 1, (p + 1) % 2)
                    pltpu.make_async_copy(
                        ego_hbm.at[colv[u]], msg[u], gsem).wait()
                    compute(p, j, u)
                    wprev = (u + NBUF - 1) % NBUF

                    @pl.when(kk > 0)
                    def _():
                        drain_scatter(wprev)

                    pltpu.async_copy(msg[u], acc.at[rowloc[u]], ssem,
                                     add=True)
                    if j == M_STEPS - 3:
                        @pl.when(bm + 1 < N_BLOCKS)
                        def _():
                            wait_macro(bm + 1, (p + 1) % 2)
                    j3 = j + 3
                    w = (u + 3) % NBUF
                    if j3 < M_STEPS:
                        fill_colv(p, j3, w)
                        pltpu.async_copy(ego_hbm.at[colv[w]], msg[w], gsem)
                    else:
                        @pl.when(bm + 1 < N_BLOCKS)
                        def _():
                            fill_colv((p + 1) % 2, j3 - M_STEPS, w)
                            pltpu.async_copy(ego_hbm.at[colv[w]], msg[w],
                                             gsem)
            return carry

        lax.fori_loop(0, N_BLOCKS // 2, block2, 0)
        drain_scatter((N_STEPS - 1) % NBUF)
        plsc.subcore_barrier()

        nch = H // CCH
        my_n = nch // NS + jnp.where(s < nch % NS, 1, 0)

        def cbody(i, carry):
            lr0 = (s + i * NS) * CCH
            gr0 = row_base + lr0
            pltpu.sync_copy(acc.at[pl.ds(lr0, CCH)], aggv)
            pltpu.sync_copy(ego_hbm.at[pl.ds(gr0, CCH)], egov)
            for g in range(CCH):
                for h in range(D // 16):
                    sl = pl.ds(h * 16, 16)
                    a = aggv[g, sl]
                    aggv[g, sl] = a + a * egov[g, sl]
            pltpu.sync_copy(aggv, out_hbm.at[pl.ds(gr0, CCH)])
            return carry

        lax.fori_loop(0, my_n, cbody, 0)

    return k(ego, gcol, grow, gval)


def _sc_gather_mean(x, e1, e2, e3, user_indices, item_indices):
    NW = NC * NS
    per_w = BATCH // NW
    GSTEP = 128

    @functools.partial(
        pl.kernel,
        out_type=[
            jax.ShapeDtypeStruct((BATCH, D), jnp.float32),
            jax.ShapeDtypeStruct((BATCH, D), jnp.float32),
        ],
        mesh=_mesh(),
        scratch_types=[
            pltpu.VMEM((GSTEP,), jnp.int32),
            pltpu.VMEM((GSTEP, D), jnp.float32),
            pltpu.VMEM((GSTEP, D), jnp.float32),
            pltpu.VMEM((GSTEP, D), jnp.float32),
            pltpu.VMEM((GSTEP, D), jnp.float32),
            pltpu.SemaphoreType.DMA,
        ],
        compiler_params=pltpu.CompilerParams(use_tc_tiling_on_sc=False),
    )
    def k(x_hbm, e1_hbm, e2_hbm, e3_hbm, ui_hbm, ii_hbm, u_out, i_out,
          idxv, b0, b1, b2, b3, sem):
        c = lax.axis_index("c")
        s = lax.axis_index("s")
        wid = s * NC + c

        def make_body(idx_hbm, out_hbm, off):
            def body(i, carry):
                r0 = wid * per_w + i * GSTEP
                pltpu.sync_copy(idx_hbm.at[pl.ds(r0, GSTEP)], idxv)
                if off:
                    for g in range(GSTEP // 16):
                        sl = pl.ds(g * 16, 16)
                        idxv[sl] = idxv[sl] + NUM_USERS
                pltpu.async_copy(x_hbm.at[idxv], b0, sem).wait()
                pltpu.async_copy(e1_hbm.at[idxv], b1, sem).wait()
                pltpu.async_copy(e2_hbm.at[idxv], b2, sem).wait()
                pltpu.async_copy(e3_hbm.at[idxv], b3, sem).wait()
                for g in range(GSTEP):
                    for h in range(D // 16):
                        sl = pl.ds(h * 16, 16)
                        b0[g, sl] = (b0[g, sl] + b1[g, sl]
                                     + b2[g, sl] + b3[g, sl]) * 0.25
                pltpu.sync_copy(b0, out_hbm.at[pl.ds(r0, GSTEP)])
                return carry
            return body

        lax.fori_loop(0, per_w // GSTEP, make_body(ui_hbm, u_out, False), 0)
        lax.fori_loop(0, per_w // GSTEP, make_body(ii_hbm, i_out, True), 0)

    return k(x, e1, e2, e3, user_indices, item_indices)


def _tc_head(u_raw, it_raw, Wa, ba, W1, b1, W2, b2):
    BLK = 2048

    def body(u_ref, i_ref, wa_ref, ba_ref, w1_ref, b1_ref, w2_ref, b2_ref,
             o_ref):
        u = u_ref[...]
        it = i_ref[...]
        u = u / jnp.maximum(
            jnp.sqrt(jnp.sum(u * u, axis=1, keepdims=True)), 1e-12)
        it = it / jnp.maximum(
            jnp.sqrt(jnp.sum(it * it, axis=1, keepdims=True)), 1e-12)
        mf = u * it
        cat = jnp.concatenate([u, it], axis=1)
        logits = jnp.dot(mf, wa_ref[...],
                         preferred_element_type=jnp.float32) + ba_ref[...]
        h = jnp.maximum(
            jnp.dot(cat, w1_ref[...],
                    preferred_element_type=jnp.float32) + b1_ref[...], 0.0)
        mlp = jnp.dot(h, w2_ref[...],
                      preferred_element_type=jnp.float32) + b2_ref[...]
        o_ref[...] = jax.nn.sigmoid(logits + mlp)

    zmap = lambda i: (0, 0)
    return pl.pallas_call(
        body,
        grid=(BATCH // BLK,),
        in_specs=[
            pl.BlockSpec((BLK, D), lambda i: (i, 0)),
            pl.BlockSpec((BLK, D), lambda i: (i, 0)),
            pl.BlockSpec((D, 1), zmap),
            pl.BlockSpec((1, 1), zmap),
            pl.BlockSpec((2 * D, 4 * D), zmap),
            pl.BlockSpec((1, 4 * D), zmap),
            pl.BlockSpec((4 * D, 1), zmap),
            pl.BlockSpec((1, 1), zmap),
        ],
        out_specs=pl.BlockSpec((BLK, 1), lambda i: (i, 0)),
        out_shape=jax.ShapeDtypeStruct((BATCH, 1), jnp.float32),
    )(u_raw, it_raw, Wa, ba.reshape(1, 1), W1, b1.reshape(1, 4 * D), W2,
      b2.reshape(1, 1))


def kernel(user_emb, item_emb, graph_val, Wa, ba, W1, b1, W2, b2,
           graph_idx, user_indices, item_indices):
    xu = _tc_normalize(user_emb)
    xi = _tc_normalize(item_emb)
    x = jnp.concatenate([xu, xi], axis=0)
    gcol = graph_idx[1]
    grow = graph_idx[0]
    e1 = _sc_layer(x, gcol, grow, graph_val)
    e2 = _sc_layer(e1, gcol, grow, graph_val)
    e3 = _sc_layer(e2, gcol, grow, graph_val)
    u_raw, it_raw = _sc_gather_mean(x, e1, e2, e3, user_indices, item_indices)
    return _tc_head(u_raw, it_raw, Wa, ba, W1, b1, W2, b2)


# speedup vs baseline: 1.1010x; 1.0190x over previous
import functools

import jax
import jax.numpy as jnp
from jax import lax
from jax.experimental import pallas as pl
from jax.experimental.pallas import tpu as pltpu
import jax.experimental.pallas.tpu_sc as plsc

NUM_USERS = 50000
NUM_ITEMS = 50000
N = NUM_USERS + NUM_ITEMS
D = 32
NNZ = 1600000
BATCH = 16384

NC = 2
NS = 16
H = N // NC
ACC_ROWS = 51200
DUMMY = H
E_TILE = NNZ // NS
E_STEP = 80
N_STEPS = E_TILE // E_STEP
NBUF = 5
M_STEPS = 5
MB = M_STEPS * E_STEP
N_BLOCKS = N_STEPS // M_STEPS
ZCH = 64
CCH = 40

_mesh = lambda: plsc.VectorSubcoreMesh(
    core_axis_name="c", subcore_axis_name="s", num_cores=NC, num_subcores=NS)


def _tc_normalize(user_emb, item_emb):
    BLK = 2000
    HB = NUM_USERS // BLK

    def body(u_ref, i_ref, o_ref):
        gi = pl.program_id(0)
        v = jnp.where(gi < HB, u_ref[...], i_ref[...])
        n = jnp.sqrt(jnp.sum(v * v, axis=1, keepdims=True))
        o_ref[...] = v / jnp.maximum(n, 1e-12)

    return pl.pallas_call(
        body,
        grid=(N // BLK,),
        in_specs=[
            pl.BlockSpec((BLK, D), lambda i: (jnp.minimum(i, HB - 1), 0)),
            pl.BlockSpec((BLK, D),
                         lambda i: (jnp.maximum(i - HB, 0), 0)),
        ],
        out_specs=pl.BlockSpec((BLK, D), lambda i: (i, 0)),
        out_shape=jax.ShapeDtypeStruct((N, D), jnp.float32),
    )(user_emb, item_emb)


def _sc_layer(ego, gcol, grow, gval):
    @functools.partial(
        pl.kernel,
        out_type=jax.ShapeDtypeStruct((N, D), jnp.float32),
        mesh=_mesh(),
        scratch_types=(
            [pltpu.VMEM((E_STEP,), jnp.int32)] * NBUF
            + [pltpu.VMEM((E_STEP,), jnp.int32)] * NBUF
            + [pltpu.VMEM((E_STEP, D), jnp.float32)] * NBUF
            + [pltpu.VMEM((MB,), jnp.int32)] * 2
            + [pltpu.VMEM((MB,), jnp.int32)] * 2
            + [pltpu.VMEM((MB,), jnp.float32)] * 2
            + [
                pltpu.VMEM((ZCH, D), jnp.float32),
                pltpu.VMEM((CCH, D), jnp.float32),
                pltpu.VMEM((CCH, D), jnp.float32),
                pltpu.VMEM_SHARED((ACC_ROWS, D), jnp.float32),
                pltpu.SemaphoreType.DMA,
                pltpu.SemaphoreType.DMA,
                pltpu.SemaphoreType.DMA,
            ]
        ),
        compiler_params=pltpu.CompilerParams(use_tc_tiling_on_sc=False),
    )
    def k(ego_hbm, gcol_hbm, grow_hbm, gval_hbm, out_hbm,
          c0, c1, c2, c3, c4, l0, l1, l2, l3, l4,
          m0, m1, m2, m3, m4, cb0, cb1, rb0, rb1, vb0, vb1,
          zv, aggv, egov, acc, gsem, msem, ssem):
        colv = [c0, c1, c2, c3, c4]
        rowloc = [l0, l1, l2, l3, l4]
        msg = [m0, m1, m2, m3, m4]
        colB = [cb0, cb1]
        rowB = [rb0, rb1]
        valB = [vb0, vb1]
        c = lax.axis_index("c")
        s = lax.axis_index("s")
        row_base = c * H

        zero = jnp.zeros((16,), jnp.float32)
        for g in range(ZCH):
            for h in range(D // 16):
                zv[g, pl.ds(h * 16, 16)] = zero
        rows_per_tile = ACC_ROWS // NS

        def zbody(i, carry):
            r0 = s * rows_per_tile + i * ZCH
            pltpu.sync_copy(zv, acc.at[pl.ds(r0, ZCH)])
            return carry

        lax.fori_loop(0, rows_per_tile // ZCH, zbody, 0)
        plsc.subcore_barrier()

        def macro_src(bm, p):
            e0 = s * E_TILE + bm * MB
            return (gcol_hbm.at[pl.ds(e0, MB)],
                    grow_hbm.at[pl.ds(e0, MB)],
                    gval_hbm.at[pl.ds(e0, MB)])

        def issue_macro(bm, p):
            cs, rs, vs = macro_src(bm, p)
            pltpu.async_copy(cs, colB[p], msem)
            pltpu.async_copy(rs, rowB[p], msem)
            pltpu.async_copy(vs, valB[p], msem)

        def wait_macro(bm, p):
            cs, rs, vs = macro_src(bm, p)
            pltpu.make_async_copy(cs, colB[p], msem).wait()
            pltpu.make_async_copy(rs, rowB[p], msem).wait()
            pltpu.make_async_copy(vs, valB[p], msem).wait()

        def fill_colv(p, j, w):
            for g in range(E_STEP // 16):
                colv[w][pl.ds(g * 16, 16)] = (
                    colB[p][pl.ds(j * E_STEP + g * 16, 16)])

        def drain_scatter(b):
            pltpu.make_async_copy(msg[b], acc.at[rowloc[b]], ssem).wait()

        def compute(p, j, u):
            mref = msg[u]
            for g in range(E_STEP // 16):
                sl = pl.ds(j * E_STEP + g * 16, 16)
                r = rowB[p][sl]
                lr = r - row_base
                ok = (lr >= 0) & (lr < H)
                dummy = DUMMY + s * 16 + lax.iota(jnp.int32, 16)
                rowloc[u][pl.ds(g * 16, 16)] = jnp.where(ok, lr, dummy)
                vv = valB[p][sl]
                for e in range(16):
                    v = vv[e]
                    idx = g * 16 + e
                    mref[idx, pl.ds(0, 16)] = mref[idx, pl.ds(0, 16)] * v
                    mref[idx, pl.ds(16, 16)] = mref[idx, pl.ds(16, 16)] * v

        cs0, rs0, vs0 = macro_src(0, 0)
        pltpu.sync_copy(cs0, colB[0])
        pltpu.sync_copy(rs0, rowB[0])
        pltpu.sync_copy(vs0, valB[0])
        for j0 in range(3):
            fill_colv(0, j0, j0)
            pltpu.async_copy(ego_hbm.at[colv[j0]], msg[j0], gsem)

        def block2(i2, carry):
            for p in (0, 1):
                bm = i2 * 2 + p
                for j in range(M_STEPS):
                    u = j % NBUF
                    kk = bm * M_STEPS + j
                    if j == 0:
                        @pl.when(bm + 1 < N_BLOCKS)
                        def _():
                            issue_macro(bm + 1, (p + 1) % 2)
                    pltpu.make_async_copy(
                        ego_hbm.at[colv[u]], msg[u], gsem).wait()
                    compute(p, j, u)
                    wprev = (u + NBUF - 1) % NBUF

                    @pl.when(kk > 0)
                    def _():
                        drain_scatter(wprev)

                    pltpu.async_copy(msg[u], acc.at[rowloc[u]], ssem,
                                     add=True)
                    if j == M_STEPS - 3:
                        @pl.when(bm + 1 < N_BLOCKS)
                        def _():
                            wait_macro(bm + 1, (p + 1) % 2)
                    j3 = j + 3
                    w = (u + 3) % NBUF
                    if j3 < M_STEPS:
                        fill_colv(p, j3, w)
                        pltpu.async_copy(ego_hbm.at[colv[w]], msg[w], gsem)
                    else:
                        @pl.when(bm + 1 < N_BLOCKS)
                        def _():
                            fill_colv((p + 1) % 2, j3 - M_STEPS, w)
                            pltpu.async_copy(ego_hbm.at[colv[w]], msg[w],
                                             gsem)
            return carry

        lax.fori_loop(0, N_BLOCKS // 2, block2, 0)
        drain_scatter((N_STEPS - 1) % NBUF)
        plsc.subcore_barrier()

        nch = H // CCH
        my_n = nch // NS + jnp.where(s < nch % NS, 1, 0)

        def cbody(i, carry):
            lr0 = (s + i * NS) * CCH
            gr0 = row_base + lr0
            pltpu.sync_copy(acc.at[pl.ds(lr0, CCH)], aggv)
            pltpu.sync_copy(ego_hbm.at[pl.ds(gr0, CCH)], egov)
            for g in range(CCH):
                for h in range(D // 16):
                    sl = pl.ds(h * 16, 16)
                    a = aggv[g, sl]
                    aggv[g, sl] = a + a * egov[g, sl]
            pltpu.sync_copy(aggv, out_hbm.at[pl.ds(gr0, CCH)])
            return carry

        lax.fori_loop(0, my_n, cbody, 0)

    return k(ego, gcol, grow, gval)


def _sc_gather_mean(x, e1, e2, e3, user_indices, item_indices):
    NW = NC * NS
    per_w = BATCH // NW
    GSTEP = 128

    @functools.partial(
        pl.kernel,
        out_type=[
            jax.ShapeDtypeStruct((BATCH, D), jnp.float32),
            jax.ShapeDtypeStruct((BATCH, D), jnp.float32),
        ],
        mesh=_mesh(),
        scratch_types=[
            pltpu.VMEM((GSTEP,), jnp.int32),
            pltpu.VMEM((GSTEP, D), jnp.float32),
            pltpu.VMEM((GSTEP, D), jnp.float32),
            pltpu.VMEM((GSTEP, D), jnp.float32),
            pltpu.VMEM((GSTEP, D), jnp.float32),
            pltpu.SemaphoreType.DMA,
        ],
        compiler_params=pltpu.CompilerParams(use_tc_tiling_on_sc=False),
    )
    def k(x_hbm, e1_hbm, e2_hbm, e3_hbm, ui_hbm, ii_hbm, u_out, i_out,
          idxv, b0, b1, b2, b3, sem):
        c = lax.axis_index("c")
        s = lax.axis_index("s")
        wid = s * NC + c

        def make_body(idx_hbm, out_hbm, off):
            def body(i, carry):
                r0 = wid * per_w + i * GSTEP
                pltpu.sync_copy(idx_hbm.at[pl.ds(r0, GSTEP)], idxv)
                if off:
                    for g in range(GSTEP // 16):
                        sl = pl.ds(g * 16, 16)
                        idxv[sl] = idxv[sl] + NUM_USERS
                pltpu.async_copy(x_hbm.at[idxv], b0, sem).wait()
                pltpu.async_copy(e1_hbm.at[idxv], b1, sem).wait()
                pltpu.async_copy(e2_hbm.at[idxv], b2, sem).wait()
                pltpu.async_copy(e3_hbm.at[idxv], b3, sem).wait()
                for g in range(GSTEP):
                    for h in range(D // 16):
                        sl = pl.ds(h * 16, 16)
                        b0[g, sl] = (b0[g, sl] + b1[g, sl]
                                     + b2[g, sl] + b3[g, sl]) * 0.25
                pltpu.sync_copy(b0, out_hbm.at[pl.ds(r0, GSTEP)])
                return carry
            return body

        lax.fori_loop(0, per_w // GSTEP, make_body(ui_hbm, u_out, False), 0)
        lax.fori_loop(0, per_w // GSTEP, make_body(ii_hbm, i_out, True), 0)

    return k(x, e1, e2, e3, user_indices, item_indices)


def _tc_head(u_raw, it_raw, Wa, ba, W1, b1, W2, b2):
    BLK = 2048

    def body(u_ref, i_ref, wa_ref, ba_ref, w1_ref, b1_ref, w2_ref, b2_ref,
             o_ref):
        u = u_ref[...]
        it = i_ref[...]
        u = u / jnp.maximum(
            jnp.sqrt(jnp.sum(u * u, axis=1, keepdims=True)), 1e-12)
        it = it / jnp.maximum(
            jnp.sqrt(jnp.sum(it * it, axis=1, keepdims=True)), 1e-12)
        mf = u * it
        cat = jnp.concatenate([u, it], axis=1)
        logits = jnp.dot(mf, wa_ref[...],
                         preferred_element_type=jnp.float32) + ba_ref[...]
        h = jnp.maximum(
            jnp.dot(cat, w1_ref[...],
                    preferred_element_type=jnp.float32) + b1_ref[...], 0.0)
        mlp = jnp.dot(h, w2_ref[...],
                      preferred_element_type=jnp.float32) + b2_ref[...]
        o_ref[...] = jax.nn.sigmoid(logits + mlp)

    zmap = lambda i: (0, 0)
    return pl.pallas_call(
        body,
        grid=(BATCH // BLK,),
        in_specs=[
            pl.BlockSpec((BLK, D), lambda i: (i, 0)),
            pl.BlockSpec((BLK, D), lambda i: (i, 0)),
            pl.BlockSpec((D, 1), zmap),
            pl.BlockSpec((1, 1), zmap),
            pl.BlockSpec((2 * D, 4 * D), zmap),
            pl.BlockSpec((1, 4 * D), zmap),
            pl.BlockSpec((4 * D, 1), zmap),
            pl.BlockSpec((1, 1), zmap),
        ],
        out_specs=pl.BlockSpec((BLK, 1), lambda i: (i, 0)),
        out_shape=jax.ShapeDtypeStruct((BATCH, 1), jnp.float32),
    )(u_raw, it_raw, Wa, ba.reshape(1, 1), W1, b1.reshape(1, 4 * D), W2,
      b2.reshape(1, 1))


def kernel(user_emb, item_emb, graph_val, Wa, ba, W1, b1, W2, b2,
           graph_idx, user_indices, item_indices):
    x = _tc_normalize(user_emb, item_emb)
    gcol = graph_idx[1]
    grow = graph_idx[0]
    e1 = _sc_layer(x, gcol, grow, graph_val)
    e2 = _sc_layer(e1, gcol, grow, graph_val)
    e3 = _sc_layer(e2, gcol, grow, graph_val)
    u_raw, it_raw = _sc_gather_mean(x, e1, e2, e3, user_indices, item_indices)
    return _tc_head(u_raw, it_raw, Wa, ba, W1, b1, W2, b2)


# speedup vs baseline: 1.1354x; 1.0312x over previous
import functools

import jax
import jax.numpy as jnp
from jax import lax
from jax.experimental import pallas as pl
from jax.experimental.pallas import tpu as pltpu
import jax.experimental.pallas.tpu_sc as plsc

NUM_USERS = 50000
NUM_ITEMS = 50000
N = NUM_USERS + NUM_ITEMS
D = 32
NNZ = 1600000
BATCH = 16384

NC = 2
NS = 16
H = N // NC
ACC_ROWS = 51200
DUMMY = H
E_TILE = NNZ // NS
E_STEP = 80
N_STEPS = E_TILE // E_STEP
NBUF = 5
M_STEPS = 5
MB = M_STEPS * E_STEP
N_BLOCKS = N_STEPS // M_STEPS
ZCH = 64
CCH = 80

_mesh = lambda: plsc.VectorSubcoreMesh(
    core_axis_name="c", subcore_axis_name="s", num_cores=NC, num_subcores=NS)


def _tc_normalize(user_emb, item_emb):
    BLK = 2000
    HB = NUM_USERS // BLK

    def body(u_ref, i_ref, o_ref):
        gi = pl.program_id(0)
        v = jnp.where(gi < HB, u_ref[...], i_ref[...])
        n = jnp.sqrt(jnp.sum(v * v, axis=1, keepdims=True))
        o_ref[...] = v / jnp.maximum(n, 1e-12)

    return pl.pallas_call(
        body,
        grid=(N // BLK,),
        in_specs=[
            pl.BlockSpec((BLK, D), lambda i: (jnp.minimum(i, HB - 1), 0)),
            pl.BlockSpec((BLK, D),
                         lambda i: (jnp.maximum(i - HB, 0), 0)),
        ],
        out_specs=pl.BlockSpec((BLK, D), lambda i: (i, 0)),
        out_shape=jax.ShapeDtypeStruct((N, D), jnp.float32),
    )(user_emb, item_emb)


def _sc_layer(ego, gcol, grow, gval):
    @functools.partial(
        pl.kernel,
        out_type=jax.ShapeDtypeStruct((N, D), jnp.float32),
        mesh=_mesh(),
        scratch_types=(
            [pltpu.VMEM((E_STEP,), jnp.int32)] * NBUF
            + [pltpu.VMEM((E_STEP,), jnp.int32)] * NBUF
            + [pltpu.VMEM((E_STEP, D), jnp.float32)] * NBUF
            + [pltpu.VMEM((MB,), jnp.int32)] * 2
            + [pltpu.VMEM((MB,), jnp.int32)] * 2
            + [pltpu.VMEM((MB,), jnp.float32)] * 2
            + [
                pltpu.VMEM((ZCH, D), jnp.float32),
                pltpu.VMEM((CCH, D), jnp.float32),
                pltpu.VMEM((CCH, D), jnp.float32),
                pltpu.VMEM_SHARED((ACC_ROWS, D), jnp.float32),
                pltpu.SemaphoreType.DMA,
                pltpu.SemaphoreType.DMA,
                pltpu.SemaphoreType.DMA,
            ]
        ),
        compiler_params=pltpu.CompilerParams(use_tc_tiling_on_sc=False),
    )
    def k(ego_hbm, gcol_hbm, grow_hbm, gval_hbm, out_hbm,
          c0, c1, c2, c3, c4, l0, l1, l2, l3, l4,
          m0, m1, m2, m3, m4, cb0, cb1, rb0, rb1, vb0, vb1,
          zv, aggv, egov, acc, gsem, msem, ssem):
        colv = [c0, c1, c2, c3, c4]
        rowloc = [l0, l1, l2, l3, l4]
        msg = [m0, m1, m2, m3, m4]
        colB = [cb0, cb1]
        rowB = [rb0, rb1]
        valB = [vb0, vb1]
        c = lax.axis_index("c")
        s = lax.axis_index("s")
        row_base = c * H

        zero = jnp.zeros((16,), jnp.float32)
        for g in range(ZCH):
            for h in range(D // 16):
                zv[g, pl.ds(h * 16, 16)] = zero
        rows_per_tile = ACC_ROWS // NS

        def zbody(i, carry):
            r0 = s * rows_per_tile + i * ZCH
            pltpu.sync_copy(zv, acc.at[pl.ds(r0, ZCH)])
            return carry

        lax.fori_loop(0, rows_per_tile // ZCH, zbody, 0)
        plsc.subcore_barrier()

        def macro_src(bm, p):
            e0 = s * E_TILE + bm * MB
            return (gcol_hbm.at[pl.ds(e0, MB)],
                    grow_hbm.at[pl.ds(e0, MB)],
                    gval_hbm.at[pl.ds(e0, MB)])

        def issue_macro(bm, p):
            cs, rs, vs = macro_src(bm, p)
            pltpu.async_copy(cs, colB[p], msem)
            pltpu.async_copy(rs, rowB[p], msem)
            pltpu.async_copy(vs, valB[p], msem)

        def wait_macro(bm, p):
            cs, rs, vs = macro_src(bm, p)
            pltpu.make_async_copy(cs, colB[p], msem).wait()
            pltpu.make_async_copy(rs, rowB[p], msem).wait()
            pltpu.make_async_copy(vs, valB[p], msem).wait()

        def fill_colv(p, j, w):
            for g in range(E_STEP // 16):
                colv[w][pl.ds(g * 16, 16)] = (
                    colB[p][pl.ds(j * E_STEP + g * 16, 16)])

        def drain_scatter(b):
            pltpu.make_async_copy(msg[b], acc.at[rowloc[b]], ssem).wait()

        def compute(p, j, u):
            mref = msg[u]
            for g in range(E_STEP // 16):
                sl = pl.ds(j * E_STEP + g * 16, 16)
                r = rowB[p][sl]
                lr = r - row_base
                ok = (lr >= 0) & (lr < H)
                dummy = DUMMY + s * 16 + lax.iota(jnp.int32, 16)
                rowloc[u][pl.ds(g * 16, 16)] = jnp.where(ok, lr, dummy)
                vv = valB[p][sl]
                for e in range(16):
                    v = vv[e]
                    idx = g * 16 + e
                    mref[idx, pl.ds(0, 16)] = mref[idx, pl.ds(0, 16)] * v
                    mref[idx, pl.ds(16, 16)] = mref[idx, pl.ds(16, 16)] * v

        cs0, rs0, vs0 = macro_src(0, 0)
        pltpu.sync_copy(cs0, colB[0])
        pltpu.sync_copy(rs0, rowB[0])
        pltpu.sync_copy(vs0, valB[0])
        for j0 in range(3):
            fill_colv(0, j0, j0)
            pltpu.async_copy(ego_hbm.at[colv[j0]], msg[j0], gsem)

        def block2(i2, carry):
            for p in (0, 1):
                bm = i2 * 2 + p
                for j in range(M_STEPS):
                    u = j % NBUF
                    kk = bm * M_STEPS + j
                    if j == 0:
                        @pl.when(bm + 1 < N_BLOCKS)
                        def _():
                            issue_macro(bm + 1, (p + 1) % 2)
                    pltpu.make_async_copy(
                        ego_hbm.at[colv[u]], msg[u], gsem).wait()
                    compute(p, j, u)
                    wprev = (u + NBUF - 1) % NBUF

                    @pl.when(kk > 0)
                    def _():
                        drain_scatter(wprev)

                    pltpu.async_copy(msg[u], acc.at[rowloc[u]], ssem,
                                     add=True)
                    if j == M_STEPS - 3:
                        @pl.when(bm + 1 < N_BLOCKS)
                        def _():
                            wait_macro(bm + 1, (p + 1) % 2)
                    j3 = j + 3
                    w = (u + 3) % NBUF
                    if j3 < M_STEPS:
                        fill_colv(p, j3, w)
                        pltpu.async_copy(ego_hbm.at[colv[w]], msg[w], gsem)
                    else:
                        @pl.when(bm + 1 < N_BLOCKS)
                        def _():
                            fill_colv((p + 1) % 2, j3 - M_STEPS, w)
                            pltpu.async_copy(ego_hbm.at[colv[w]], msg[w],
                                             gsem)
            return carry

        lax.fori_loop(0, N_BLOCKS // 2, block2, 0)
        drain_scatter((N_STEPS - 1) % NBUF)
        plsc.subcore_barrier()

        nch = H // CCH
        my_n = nch // NS + jnp.where(s < nch % NS, 1, 0)

        def cbody(i, carry):
            lr0 = (s + i * NS) * CCH
            gr0 = row_base + lr0
            pltpu.sync_copy(acc.at[pl.ds(lr0, CCH)], aggv)
            pltpu.sync_copy(ego_hbm.at[pl.ds(gr0, CCH)], egov)
            for g in range(CCH):
                for h in range(D // 16):
                    sl = pl.ds(h * 16, 16)
                    a = aggv[g, sl]
                    aggv[g, sl] = a + a * egov[g, sl]
            pltpu.sync_copy(aggv, out_hbm.at[pl.ds(gr0, CCH)])
            return carry

        lax.fori_loop(0, my_n, cbody, 0)

    return k(ego, gcol, grow, gval)


def _sc_gather_mean(x, e1, e2, e3, user_indices, item_indices):
    NW = NC * NS
    per_w = BATCH // NW
    GSTEP = 128

    @functools.partial(
        pl.kernel,
        out_type=[
            jax.ShapeDtypeStruct((BATCH, D), jnp.float32),
            jax.ShapeDtypeStruct((BATCH, D), jnp.float32),
        ],
        mesh=_mesh(),
        scratch_types=[
            pltpu.VMEM((GSTEP,), jnp.int32),
            pltpu.VMEM((GSTEP, D), jnp.float32),
            pltpu.VMEM((GSTEP, D), jnp.float32),
            pltpu.VMEM((GSTEP, D), jnp.float32),
            pltpu.VMEM((GSTEP, D), jnp.float32),
            pltpu.SemaphoreType.DMA,
        ],
        compiler_params=pltpu.CompilerParams(use_tc_tiling_on_sc=False),
    )
    def k(x_hbm, e1_hbm, e2_hbm, e3_hbm, ui_hbm, ii_hbm, u_out, i_out,
          idxv, b0, b1, b2, b3, sem):
        c = lax.axis_index("c")
        s = lax.axis_index("s")
        wid = s * NC + c

        def make_body(idx_hbm, out_hbm, off):
            def body(i, carry):
                r0 = wid * per_w + i * GSTEP
                pltpu.sync_copy(idx_hbm.at[pl.ds(r0, GSTEP)], idxv)
                if off:
                    for g in range(GSTEP // 16):
                        sl = pl.ds(g * 16, 16)
                        idxv[sl] = idxv[sl] + NUM_USERS
                pltpu.async_copy(x_hbm.at[idxv], b0, sem).wait()
                pltpu.async_copy(e1_hbm.at[idxv], b1, sem).wait()
                pltpu.async_copy(e2_hbm.at[idxv], b2, sem).wait()
                pltpu.async_copy(e3_hbm.at[idxv], b3, sem).wait()
                for g in range(GSTEP):
                    for h in range(D // 16):
                        sl = pl.ds(h * 16, 16)
                        b0[g, sl] = (b0[g, sl] + b1[g, sl]
                                     + b2[g, sl] + b3[g, sl]) * 0.25
                pltpu.sync_copy(b0, out_hbm.at[pl.ds(r0, GSTEP)])
                return carry
            return body

        lax.fori_loop(0, per_w // GSTEP, make_body(ui_hbm, u_out, False), 0)
        lax.fori_loop(0, per_w // GSTEP, make_body(ii_hbm, i_out, True), 0)

    return k(x, e1, e2, e3, user_indices, item_indices)


def _tc_head(u_raw, it_raw, Wa, ba, W1, b1, W2, b2):
    BLK = 2048

    def body(u_ref, i_ref, wa_ref, ba_ref, w1_ref, b1_ref, w2_ref, b2_ref,
             o_ref):
        u = u_ref[...]
        it = i_ref[...]
        u = u / jnp.maximum(
            jnp.sqrt(jnp.sum(u * u, axis=1, keepdims=True)), 1e-12)
        it = it / jnp.maximum(
            jnp.sqrt(jnp.sum(it * it, axis=1, keepdims=True)), 1e-12)
        mf = u * it
        cat = jnp.concatenate([u, it], axis=1)
        logits = jnp.dot(mf, wa_ref[...],
                         preferred_element_type=jnp.float32) + ba_ref[...]
        h = jnp.maximum(
            jnp.dot(cat, w1_ref[...],
                    preferred_element_type=jnp.float32) + b1_ref[...], 0.0)
        mlp = jnp.dot(h, w2_ref[...],
                      preferred_element_type=jnp.float32) + b2_ref[...]
        o_ref[...] = jax.nn.sigmoid(logits + mlp)

    zmap = lambda i: (0, 0)
    return pl.pallas_call(
        body,
        grid=(BATCH // BLK,),
        in_specs=[
            pl.BlockSpec((BLK, D), lambda i: (i, 0)),
            pl.BlockSpec((BLK, D), lambda i: (i, 0)),
            pl.BlockSpec((D, 1), zmap),
            pl.BlockSpec((1, 1), zmap),
            pl.BlockSpec((2 * D, 4 * D), zmap),
            pl.BlockSpec((1, 4 * D), zmap),
            pl.BlockSpec((4 * D, 1), zmap),
            pl.BlockSpec((1, 1), zmap),
        ],
        out_specs=pl.BlockSpec((BLK, 1), lambda i: (i, 0)),
        out_shape=jax.ShapeDtypeStruct((BATCH, 1), jnp.float32),
    )(u_raw, it_raw, Wa, ba.reshape(1, 1), W1, b1.reshape(1, 4 * D), W2,
      b2.reshape(1, 1))


def kernel(user_emb, item_emb, graph_val, Wa, ba, W1, b1, W2, b2,
           graph_idx, user_indices, item_indices):
    x = _tc_normalize(user_emb, item_emb)
    gcol = graph_idx[1]
    grow = graph_idx[0]
    e1 = _sc_layer(x, gcol, grow, graph_val)
    e2 = _sc_layer(e1, gcol, grow, graph_val)
    e3 = _sc_layer(e2, gcol, grow, graph_val)
    u_raw, it_raw = _sc_gather_mean(x, e1, e2, e3, user_indices, item_indices)
    return _tc_head(u_raw, it_raw, Wa, ba, W1, b1, W2, b2)
